# Initial kernel scaffold; baseline (speedup 1.0000x reference)
import jax
import jax.numpy as jnp
from jax.experimental import pallas as pl


def kernel(x, edge_index, W1, b1, W2, b2):
    raise NotImplementedError("write your pallas kernel here")


# speedup vs baseline: 100.5457x; 100.5457x over previous
import jax
import jax.numpy as jnp
from jax import lax
from jax.experimental import pallas as pl
from jax.experimental.pallas import tpu as pltpu
from jax.experimental.pallas import tpu_sc as plsc

_NT = 16
_L = 16


def _make_sc_kernel(G, Gp, Ep):
    EPT = Ep // _NT
    GPT = Gp // _NT
    mesh = plsc.VectorSubcoreMesh(
        core_axis_name="c", subcore_axis_name="s", num_cores=1)

    def body(src_h, dst_h, x0_h, s1_h, wgt_h,
             src_v, dst_v, x0_v, dinv_v, hist_v, t_v, c_v, red_v, o1_v, o2_v,
             sh_a, sh_b, sh_dinv):
        sid = lax.axis_index("s")
        ebase = sid * EPT
        gbase = sid * GPT

        pltpu.sync_copy(src_h.at[pl.ds(ebase, EPT)], src_v)
        pltpu.sync_copy(dst_h.at[pl.ds(ebase, EPT)], dst_v)
        pltpu.sync_copy(x0_h, x0_v)

        zeros = jnp.zeros((_L,), jnp.float32)

        def zbody(i, c):
            hist_v[pl.ds(i * _L, _L)] = zeros
            t_v[pl.ds(i * _L, _L)] = zeros
            c_v[pl.ds(i * _L, _L)] = zeros
            return c
        lax.fori_loop(0, Gp // _L, zbody, 0)

        ones = jnp.ones((_L,), jnp.float32)

        def hbody(i, c):
            d = dst_v[pl.ds(i * _L, _L)]
            plsc.addupdate_scatter(hist_v, [d], ones)
            return c
        lax.fori_loop(0, EPT // _L, hbody, 0)

        pltpu.sync_copy(hist_v, sh_a.at[sid])
        plsc.subcore_barrier()

        for k in range(_NT):
            pltpu.sync_copy(sh_a.at[k, pl.ds(gbase, GPT)], red_v.at[k])

        def dbody(i, c):
            acc = red_v[0, pl.ds(i * _L, _L)]
            for k in range(1, _NT):
                acc = acc + red_v[k, pl.ds(i * _L, _L)]
            deg = acc + 1.0
            bits = plsc.bitcast(deg, jnp.int32)
            y = plsc.bitcast(
                jnp.int32(0x5F3759DF) - lax.shift_right_logical(bits, 1),
                jnp.float32)
            for _ in range(3):
                y = y * (1.5 - 0.5 * deg * y * y)
            dinv_v[pl.ds(gbase + i * _L, _L)] = y
            return c
        lax.fori_loop(0, GPT // _L, dbody, 0)

        pltpu.sync_copy(dinv_v.at[pl.ds(gbase, GPT)], sh_dinv.at[pl.ds(gbase, GPT)])
        plsc.subcore_barrier()
        pltpu.sync_copy(sh_dinv, dinv_v)

        def ebody(i, c):
            s = src_v[pl.ds(i * _L, _L)]
            d = dst_v[pl.ds(i * _L, _L)]
            dv_s = plsc.load_gather(dinv_v, [s])
            dv_d = plsc.load_gather(dinv_v, [d])
            xs = plsc.load_gather(x0_v, [s])
            plsc.addupdate_scatter(t_v, [d], xs * dv_s)
            plsc.addupdate_scatter(c_v, [s], dv_d)
            return c
        lax.fori_loop(0, EPT // _L, ebody, 0)

        pltpu.sync_copy(t_v, sh_a.at[sid])
        pltpu.sync_copy(c_v, sh_b.at[sid])
        plsc.subcore_barrier()

        for k in range(_NT):
            pltpu.sync_copy(sh_a.at[k, pl.ds(gbase, GPT)], red_v.at[k])

        def f1body(i, c):
            acc = red_v[0, pl.ds(i * _L, _L)]
            for k in range(1, _NT):
                acc = acc + red_v[k, pl.ds(i * _L, _L)]
            dv = dinv_v[pl.ds(gbase + i * _L, _L)]
            xv = x0_v[pl.ds(gbase + i * _L, _L)]
            o1_v[pl.ds(i * _L, _L)] = dv * acc + dv * dv * xv
            return c
        lax.fori_loop(0, GPT // _L, f1body, 0)

        for k in range(_NT):
            pltpu.sync_copy(sh_b.at[k, pl.ds(gbase, GPT)], red_v.at[k])

        lane = lax.iota(jnp.int32, _L)

        def f2body(i, c):
            acc = red_v[0, pl.ds(i * _L, _L)]
            for k in range(1, _NT):
                acc = acc + red_v[k, pl.ds(i * _L, _L)]
            dv = dinv_v[pl.ds(gbase + i * _L, _L)]
            w = dv * acc + dv * dv
            gidx = gbase + i * _L + lane
            w = jnp.where(gidx < G, w, 0.0)
            o2_v[pl.ds(i * _L, _L)] = w
            return c
        lax.fori_loop(0, GPT // _L, f2body, 0)

        pltpu.sync_copy(o1_v, s1_h.at[pl.ds(gbase, GPT)])
        pltpu.sync_copy(o2_v, wgt_h.at[pl.ds(gbase, GPT)])

    return pl.kernel(
        body,
        out_type=(jax.ShapeDtypeStruct((Gp,), jnp.float32),
                  jax.ShapeDtypeStruct((Gp,), jnp.float32)),
        mesh=mesh,
        compiler_params=pltpu.CompilerParams(needs_layout_passes=False),
        scratch_types=[
            pltpu.VMEM((EPT,), jnp.int32),
            pltpu.VMEM((EPT,), jnp.int32),
            pltpu.VMEM((Gp,), jnp.float32),
            pltpu.VMEM((Gp,), jnp.float32),
            pltpu.VMEM((Gp,), jnp.float32),
            pltpu.VMEM((Gp,), jnp.float32),
            pltpu.VMEM((Gp,), jnp.float32),
            pltpu.VMEM((_NT, GPT), jnp.float32),
            pltpu.VMEM((GPT,), jnp.float32),
            pltpu.VMEM((GPT,), jnp.float32),
            pltpu.VMEM_SHARED((_NT, Gp), jnp.float32),
            pltpu.VMEM_SHARED((_NT, Gp), jnp.float32),
            pltpu.VMEM_SHARED((Gp,), jnp.float32),
        ],
    )


def _make_dense_kernel(B, G, Gp, HID, OUT, BLK):
    BPB = Gp // BLK

    def body(s_ref, wt_ref, w1_ref, b1_ref, w2_ref, b2_ref, out_ref, acc_ref):
        i = pl.program_id(0)
        j = i % BPB
        s = s_ref[0]
        wt = wt_ref[0]
        h = jnp.maximum(w1_ref[...] * s + b1_ref[...], 0.0)
        part = jnp.sum(h * wt, axis=1, keepdims=True)

        @pl.when(j == 0)
        def _():
            acc_ref[...] = part

        @pl.when(j != 0)
        def _():
            acc_ref[...] = acc_ref[...] + part

        @pl.when(j == BPB - 1)
        def _():
            out_ref[0] = lax.dot_general(
                acc_ref[...] * (1.0 / G), w2_ref[...],
                (((0,), (0,)), ((), ())),
                preferred_element_type=jnp.float32) + b2_ref[...]

    return pl.pallas_call(
        body,
        grid=(B * BPB,),
        in_specs=[
            pl.BlockSpec((1, 1, BLK), lambda i: (i, 0, 0)),
            pl.BlockSpec((1, 1, BLK), lambda i: (i, 0, 0)),
            pl.BlockSpec((HID, 1), lambda i: (0, 0)),
            pl.BlockSpec((HID, 1), lambda i: (0, 0)),
            pl.BlockSpec((HID, OUT), lambda i: (0, 0)),
            pl.BlockSpec((1, OUT), lambda i: (0, 0)),
        ],
        out_specs=pl.BlockSpec((1, 1, OUT), lambda i: (i // BPB, 0, 0)),
        out_shape=jax.ShapeDtypeStruct((B, 1, OUT), jnp.float32),
        scratch_shapes=[pltpu.VMEM((HID, 1), jnp.float32)],
    )


def kernel(x, edge_index, W1, b1, W2, b2):
    B, G = x.shape
    E = edge_index.shape[1]
    HID = W1.shape[1]
    OUT = W2.shape[1]

    Gp = -(-G // (_NT * _L)) * (_NT * _L)
    Ep = -(-E // (_NT * _L)) * (_NT * _L)

    src = edge_index[0]
    dst = edge_index[1]
    if Ep != E:
        pad = jnp.full((Ep - E,), Gp - 1, dtype=edge_index.dtype)
        src = jnp.concatenate([src, pad])
        dst = jnp.concatenate([dst, pad])

    x0 = jnp.pad(x[0], (0, Gp - G))
    s1, wgt = _make_sc_kernel(G, Gp, Ep)(src, dst, x0)

    xp = jnp.pad(x, ((0, 0), (0, Gp - G)))
    colmask = (jnp.arange(Gp, dtype=jnp.int32) < G).astype(jnp.float32)
    S = jnp.concatenate([s1[None], xp[1:]], axis=0)
    Wt = jnp.concatenate([wgt[None], jnp.broadcast_to(colmask, (B - 1, Gp))],
                         axis=0)

    BLK = Gp
    for bpb in (5, 4, 8, 2, 10, 16):
        if Gp % bpb == 0 and (Gp // bpb) % 128 == 0:
            BLK = Gp // bpb
            break

    BPB = Gp // BLK
    dense = _make_dense_kernel(B, G, Gp, HID, OUT, BLK)
    out = dense(S.reshape(B * BPB, 1, BLK), Wt.reshape(B * BPB, 1, BLK),
                W1.reshape(HID, 1), b1.reshape(HID, 1), W2,
                b2.reshape(1, OUT))
    return out.reshape(B, OUT)


# speedup vs baseline: 138.7267x; 1.3797x over previous
import jax
import jax.numpy as jnp
from jax import lax
from jax.experimental import pallas as pl
from jax.experimental.pallas import tpu as pltpu
from jax.experimental.pallas import tpu_sc as plsc

_NT = 16
_L = 16


def _make_sc_kernel(G, Gp, Ep):
    EPT = Ep // _NT
    GPT = Gp // _NT
    mesh = plsc.VectorSubcoreMesh(
        core_axis_name="c", subcore_axis_name="s", num_cores=1)

    def body(ei_h, x_h, s1_h, wgt_h,
             src_v, dst_v, x0_v, dinv_v, hist_v, t_v, c_v, red_v, o1_v, o2_v,
             sh_a, sh_b, sh_dinv):
        sid = lax.axis_index("s")
        ebase = sid * EPT
        gbase = sid * GPT

        pltpu.sync_copy(ei_h.at[pl.ds(ebase, EPT)], src_v)
        pltpu.sync_copy(ei_h.at[pl.ds(Ep + ebase, EPT)], dst_v)
        zeros = jnp.zeros((_L,), jnp.float32)

        def xzbody(i, c):
            x0_v[pl.ds(G + i * _L, _L)] = zeros
            return c
        lax.fori_loop(0, (Gp - G) // _L, xzbody, 0)
        pltpu.sync_copy(x_h.at[pl.ds(0, G)], x0_v.at[pl.ds(0, G)])

        def zbody(i, c):
            hist_v[pl.ds(i * _L, _L)] = zeros
            t_v[pl.ds(i * _L, _L)] = zeros
            c_v[pl.ds(i * _L, _L)] = zeros
            return c
        lax.fori_loop(0, Gp // _L, zbody, 0)

        ones = jnp.ones((_L,), jnp.float32)

        def hbody(i, c):
            d = dst_v[pl.ds(i * _L, _L)]
            plsc.addupdate_scatter(hist_v, [d], ones)
            return c
        lax.fori_loop(0, EPT // _L, hbody, 0)

        pltpu.sync_copy(hist_v, sh_a.at[sid])
        plsc.subcore_barrier()

        for k in range(_NT):
            pltpu.sync_copy(sh_a.at[k, pl.ds(gbase, GPT)], red_v.at[k])

        def dbody(i, c):
            acc = red_v[0, pl.ds(i * _L, _L)]
            for k in range(1, _NT):
                acc = acc + red_v[k, pl.ds(i * _L, _L)]
            deg = acc + 1.0
            bits = plsc.bitcast(deg, jnp.int32)
            y = plsc.bitcast(
                jnp.int32(0x5F3759DF) - lax.shift_right_logical(bits, 1),
                jnp.float32)
            for _ in range(3):
                y = y * (1.5 - 0.5 * deg * y * y)
            dinv_v[pl.ds(gbase + i * _L, _L)] = y
            return c
        lax.fori_loop(0, GPT // _L, dbody, 0)

        pltpu.sync_copy(dinv_v.at[pl.ds(gbase, GPT)], sh_dinv.at[pl.ds(gbase, GPT)])
        plsc.subcore_barrier()
        pltpu.sync_copy(sh_dinv, dinv_v)

        def ebody(i, c):
            s = src_v[pl.ds(i * _L, _L)]
            d = dst_v[pl.ds(i * _L, _L)]
            dv_s = plsc.load_gather(dinv_v, [s])
            dv_d = plsc.load_gather(dinv_v, [d])
            xs = plsc.load_gather(x0_v, [s])
            plsc.addupdate_scatter(t_v, [d], xs * dv_s)
            plsc.addupdate_scatter(c_v, [s], dv_d)
            return c
        lax.fori_loop(0, EPT // _L, ebody, 0)

        pltpu.sync_copy(t_v, sh_a.at[sid])
        pltpu.sync_copy(c_v, sh_b.at[sid])
        plsc.subcore_barrier()

        for k in range(_NT):
            pltpu.sync_copy(sh_a.at[k, pl.ds(gbase, GPT)], red_v.at[k])

        def f1body(i, c):
            acc = red_v[0, pl.ds(i * _L, _L)]
            for k in range(1, _NT):
                acc = acc + red_v[k, pl.ds(i * _L, _L)]
            dv = dinv_v[pl.ds(gbase + i * _L, _L)]
            xv = x0_v[pl.ds(gbase + i * _L, _L)]
            o1_v[pl.ds(i * _L, _L)] = dv * acc + dv * dv * xv
            return c
        lax.fori_loop(0, GPT // _L, f1body, 0)

        for k in range(_NT):
            pltpu.sync_copy(sh_b.at[k, pl.ds(gbase, GPT)], red_v.at[k])

        lane = lax.iota(jnp.int32, _L)

        def f2body(i, c):
            acc = red_v[0, pl.ds(i * _L, _L)]
            for k in range(1, _NT):
                acc = acc + red_v[k, pl.ds(i * _L, _L)]
            dv = dinv_v[pl.ds(gbase + i * _L, _L)]
            w = dv * acc + dv * dv
            gidx = gbase + i * _L + lane
            w = jnp.where(gidx < G, w, 0.0)
            o2_v[pl.ds(i * _L, _L)] = w
            return c
        lax.fori_loop(0, GPT // _L, f2body, 0)

        pltpu.sync_copy(o1_v, s1_h.at[pl.ds(gbase, GPT)])
        pltpu.sync_copy(o2_v, wgt_h.at[pl.ds(gbase, GPT)])

    return pl.kernel(
        body,
        out_type=(jax.ShapeDtypeStruct((Gp,), jnp.float32),
                  jax.ShapeDtypeStruct((Gp,), jnp.float32)),
        mesh=mesh,
        compiler_params=pltpu.CompilerParams(needs_layout_passes=False),
        scratch_types=[
            pltpu.VMEM((EPT,), jnp.int32),
            pltpu.VMEM((EPT,), jnp.int32),
            pltpu.VMEM((Gp,), jnp.float32),
            pltpu.VMEM((Gp,), jnp.float32),
            pltpu.VMEM((Gp,), jnp.float32),
            pltpu.VMEM((Gp,), jnp.float32),
            pltpu.VMEM((Gp,), jnp.float32),
            pltpu.VMEM((_NT, GPT), jnp.float32),
            pltpu.VMEM((GPT,), jnp.float32),
            pltpu.VMEM((GPT,), jnp.float32),
            pltpu.VMEM_SHARED((_NT, Gp), jnp.float32),
            pltpu.VMEM_SHARED((_NT, Gp), jnp.float32),
            pltpu.VMEM_SHARED((Gp,), jnp.float32),
        ],
    )


def _make_dense_kernel(B, G, Gp, HID, OUT):
    def body(s1_ref, wgt_ref, x_ref, w1_ref, b1_ref, w2_ref, b2_ref, out_ref):
        i = pl.program_id(0)
        w1 = w1_ref[...]
        b1 = b1_ref[...]

        @pl.when(i == 0)
        def _():
            h = jnp.maximum(w1 * s1_ref[...] + b1, 0.0)
            part = jnp.sum(h * wgt_ref[...], axis=1, keepdims=True)
            out_ref[0] = lax.dot_general(
                part * (1.0 / G), w2_ref[...],
                (((0,), (0,)), ((), ())),
                preferred_element_type=jnp.float32) + b2_ref[...]

        @pl.when(i != 0)
        def _():
            h = jnp.maximum(w1 * x_ref[0] + b1, 0.0)
            part = jnp.sum(h, axis=1, keepdims=True)
            out_ref[0] = lax.dot_general(
                part * (1.0 / G), w2_ref[...],
                (((0,), (0,)), ((), ())),
                preferred_element_type=jnp.float32) + b2_ref[...]

    return pl.pallas_call(
        body,
        grid=(B,),
        in_specs=[
            pl.BlockSpec((1, Gp), lambda i: (0, 0)),
            pl.BlockSpec((1, Gp), lambda i: (0, 0)),
            pl.BlockSpec((1, 1, G), lambda i: (i, 0, 0)),
            pl.BlockSpec((HID, 1), lambda i: (0, 0)),
            pl.BlockSpec((HID, 1), lambda i: (0, 0)),
            pl.BlockSpec((HID, OUT), lambda i: (0, 0)),
            pl.BlockSpec((1, OUT), lambda i: (0, 0)),
        ],
        out_specs=pl.BlockSpec((1, 1, OUT), lambda i: (i, 0, 0)),
        out_shape=jax.ShapeDtypeStruct((B, 1, OUT), jnp.float32),
    )


def kernel(x, edge_index, W1, b1, W2, b2):
    B, G = x.shape
    E = edge_index.shape[1]
    HID = W1.shape[1]
    OUT = W2.shape[1]

    Gp = -(-G // (_NT * _L)) * (_NT * _L)
    Ep = -(-E // (_NT * _L)) * (_NT * _L)

    ei = edge_index
    if Ep != E:
        pad = jnp.full((2, Ep - E), Gp - 1, dtype=edge_index.dtype)
        ei = jnp.concatenate([edge_index, pad], axis=1)

    s1, wgt = _make_sc_kernel(G, Gp, Ep)(ei.reshape(2 * Ep), x.reshape(B * G))

    dense = _make_dense_kernel(B, G, Gp, HID, OUT)
    out = dense(s1.reshape(1, Gp), wgt.reshape(1, Gp), x.reshape(B, 1, G),
                W1.reshape(HID, 1), b1.reshape(HID, 1), W2,
                b2.reshape(1, OUT))
    return out.reshape(B, OUT)
